# Initial kernel scaffold; baseline (speedup 1.0000x reference)
import functools

import jax
import jax.numpy as jnp
from jax import lax
from jax.experimental import pallas as pl
from jax.experimental.pallas import tpu as pltpu
from jax.experimental.pallas import tpu_sc as plsc

N = 100000
E = 1600000
IN_F = 8
HID = 32
HALF = 16
NCORE = 2
NSUB = 16

NV = N + 16
ZROWS = NV // NSUB
WROWS = N // NSUB

EP = 1638400
PAD = EP - E
ER = EP // 128

MS_SLAB = 16
MS_RPS = ER // NSUB
MS_NSLAB = MS_RPS // MS_SLAB

DG_SLAB = 8
DG_RPS = ER // (NCORE * NSUB)
DG_NSLAB = DG_RPS // DG_SLAB

BN = 2000
GRID = N // BN

_mesh = plsc.VectorSubcoreMesh(core_axis_name="c", subcore_axis_name="s")


def _deg_body(dstp, zeros, ones, degh, didx, ones_v, degS, sem_i):
  c = lax.axis_index("c")
  s = lax.axis_index("s")
  pltpu.sync_copy(zeros, degS.at[pl.ds(s * ZROWS, ZROWS)])
  pltpu.sync_copy(ones, ones_v)
  row0 = (c * NSUB + s) * DG_RPS

  def idx_start(slab, b):
    pltpu.async_copy(
        dstp.at[pl.ds(row0 + slab * DG_SLAB, DG_SLAB)], didx.at[b],
        sem_i.at[b])

  def idx_wait(slab, b):
    pltpu.make_async_copy(
        dstp.at[pl.ds(row0 + slab * DG_SLAB, DG_SLAB)], didx.at[b],
        sem_i.at[b]).wait()

  idx_start(0, 0)
  plsc.subcore_barrier()

  def outer(g, carry):
    for b in (0, 1):
      i = g * 2 + b
      idx_wait(i, b)

      @pl.when(i + 1 < DG_NSLAB)
      def _():
        idx_start(i + 1, 1 - b)

      for j in range(DG_SLAB):
        pltpu.sync_copy(ones_v, degS.at[didx.at[b, j]], add=True)
    return carry

  lax.fori_loop(0, DG_NSLAB // 2, outer, 0)
  plsc.subcore_barrier()
  pltpu.sync_copy(degS.at[pl.ds(s * WROWS, WROWS)],
                  degh.at[c, pl.ds(s * WROWS, WROWS)])


_deg_call = pl.kernel(
    _deg_body,
    out_type=jax.ShapeDtypeStruct((NCORE, N, HALF), jnp.float32),
    mesh=_mesh,
    scratch_types=[
        pltpu.VMEM((2, DG_SLAB, 128), jnp.int32),
        pltpu.VMEM((128, HALF), jnp.float32),
        pltpu.VMEM_SHARED((NV, HALF), jnp.float32),
        pltpu.SemaphoreType.DMA((2,)),
    ],
)


def _msg_body(ycat, srcs2, dstp, zeros, acc, sidx, didx, rows, accS,
              sem_g, sem_i):
  c = lax.axis_index("c")
  s = lax.axis_index("s")
  pltpu.sync_copy(zeros, accS.at[pl.ds(s * ZROWS, ZROWS)])
  row0 = s * MS_RPS

  def idx_start(slab, b):
    r0 = row0 + slab * MS_SLAB
    pltpu.async_copy(srcs2.at[c, pl.ds(r0, MS_SLAB)], sidx.at[b],
                     sem_i.at[b])
    pltpu.async_copy(dstp.at[pl.ds(r0, MS_SLAB)], didx.at[b], sem_i.at[b])

  def idx_wait(slab, b):
    r0 = row0 + slab * MS_SLAB
    pltpu.make_async_copy(srcs2.at[c, pl.ds(r0, MS_SLAB)], sidx.at[b],
                          sem_i.at[b]).wait()
    pltpu.make_async_copy(dstp.at[pl.ds(r0, MS_SLAB)], didx.at[b],
                          sem_i.at[b]).wait()

  def fire_gathers(b):
    for j in range(MS_SLAB):
      pltpu.async_copy(ycat.at[sidx.at[b, j]],
                       rows.at[b, pl.ds(j * 128, 128)], sem_g.at[b])

  def drain_gathers(b):
    for j in range(MS_SLAB):
      pltpu.make_async_copy(ycat.at[sidx.at[b, j]],
                            rows.at[b, pl.ds(j * 128, 128)],
                            sem_g.at[b]).wait()

  def scatters(b):
    for j in range(MS_SLAB):
      pltpu.sync_copy(rows.at[b, pl.ds(j * 128, 128)],
                      accS.at[didx.at[b, j]], add=True)

  idx_start(0, 0)
  idx_wait(0, 0)
  fire_gathers(0)
  idx_start(1, 1)
  plsc.subcore_barrier()

  def outer(g, carry):
    for b in (0, 1):
      i = g * 2 + b
      drain_gathers(b)

      @pl.when(i + 1 < MS_NSLAB)
      def _():
        idx_wait(i + 1, 1 - b)
        fire_gathers(1 - b)

      scatters(b)

      @pl.when(i + 2 < MS_NSLAB)
      def _():
        idx_start(i + 2, b)
    return carry

  lax.fori_loop(0, MS_NSLAB // 2, outer, 0)
  plsc.subcore_barrier()
  pltpu.sync_copy(accS.at[pl.ds(s * WROWS, WROWS)],
                  acc.at[c, pl.ds(s * WROWS, WROWS)])


_msg_call = pl.kernel(
    _msg_body,
    out_type=jax.ShapeDtypeStruct((NCORE, N, HALF), jnp.float32),
    mesh=_mesh,
    scratch_types=[
        pltpu.VMEM((2, MS_SLAB, 128), jnp.int32),
        pltpu.VMEM((2, MS_SLAB, 128), jnp.int32),
        pltpu.VMEM((2, MS_SLAB * 128, HALF), jnp.float32),
        pltpu.VMEM_SHARED((NV, HALF), jnp.float32),
        pltpu.SemaphoreType.DMA((2,)),
        pltpu.SemaphoreType.DMA((2,)),
    ],
)


def _dense1_body(x, degh, w1, ycat, self1, dish):
  deg = degh[0][:, 0:1] + degh[1][:, 0:1] + 1.0
  dis = lax.rsqrt(deg)
  xw = jnp.dot(x[...], w1[...], preferred_element_type=jnp.float32)
  y = xw * dis
  ycat[...] = jnp.concatenate([y[:, :HALF][None], y[:, HALF:][None]], axis=0)
  self1[...] = y * dis
  dish[...] = jnp.broadcast_to(dis, (BN, HALF))


def _tc_dense1(x, degh, w1):
  return pl.pallas_call(
      _dense1_body,
      grid=(GRID,),
      in_specs=[
          pl.BlockSpec((BN, IN_F), lambda i: (i, 0)),
          pl.BlockSpec((NCORE, BN, HALF), lambda i: (0, i, 0)),
          pl.BlockSpec((IN_F, HID), lambda i: (0, 0)),
      ],
      out_specs=[
          pl.BlockSpec((NCORE, BN, HALF), lambda i: (0, i, 0)),
          pl.BlockSpec((BN, HID), lambda i: (i, 0)),
          pl.BlockSpec((BN, HALF), lambda i: (i, 0)),
      ],
      out_shape=[
          jax.ShapeDtypeStruct((NCORE, N, HALF), jnp.float32),
          jax.ShapeDtypeStruct((N, HID), jnp.float32),
          jax.ShapeDtypeStruct((N, HALF), jnp.float32),
      ],
  )(x, degh, w1)


def _dense2_body(acc1, self1, dish, w2, b1, ycat2, self2):
  dis = jnp.concatenate([dish[...], dish[...]], axis=1)
  accfull = jnp.concatenate([acc1[0], acc1[1]], axis=1)
  h1 = jnp.maximum(dis * accfull + self1[...] + b1[...], 0.0)
  xw2 = jnp.dot(h1, w2[...], preferred_element_type=jnp.float32)
  y2 = xw2 * dis
  ycat2[...] = jnp.concatenate([y2[:, :HALF][None], y2[:, HALF:][None]],
                               axis=0)
  self2[...] = y2 * dis


def _tc_dense2(acc1, self1, dish, w2, b1):
  return pl.pallas_call(
      _dense2_body,
      grid=(GRID,),
      in_specs=[
          pl.BlockSpec((NCORE, BN, HALF), lambda i: (0, i, 0)),
          pl.BlockSpec((BN, HID), lambda i: (i, 0)),
          pl.BlockSpec((BN, HALF), lambda i: (i, 0)),
          pl.BlockSpec((HID, HID), lambda i: (0, 0)),
          pl.BlockSpec((1, HID), lambda i: (0, 0)),
      ],
      out_specs=[
          pl.BlockSpec((NCORE, BN, HALF), lambda i: (0, i, 0)),
          pl.BlockSpec((BN, HID), lambda i: (i, 0)),
      ],
      out_shape=[
          jax.ShapeDtypeStruct((NCORE, N, HALF), jnp.float32),
          jax.ShapeDtypeStruct((N, HID), jnp.float32),
      ],
  )(acc1, self1, dish, w2, b1)


def _final_body(acc2, self2, dish, b2, wfc, bfc, out, scr):
  i = pl.program_id(0)
  dis = jnp.concatenate([dish[...], dish[...]], axis=1)
  accfull = jnp.concatenate([acc2[0], acc2[1]], axis=1)
  h2 = jnp.maximum(dis * accfull + self2[...] + b2[...], 0.0)
  bmax = jnp.max(h2, axis=0, keepdims=True)
  prev = jnp.where(i == 0, jnp.full((1, HID), -jnp.inf, jnp.float32),
                   scr[0:1, 0:HID])
  scr[0:1, 0:HID] = jnp.maximum(bmax, prev)

  @pl.when(i == GRID - 1)
  def _():
    pooled = scr[0:1, 0:HID]
    logits = jnp.sum(pooled.reshape(HID, 1) * wfc[...], axis=0,
                     keepdims=True) + bfc[...]
    m = jnp.max(logits, axis=1, keepdims=True)
    z = logits - m
    out[...] = z - jnp.log(jnp.sum(jnp.exp(z), axis=1, keepdims=True))


def _tc_final(acc2, self2, dish, b2, wfc, bfc):
  return pl.pallas_call(
      _final_body,
      grid=(GRID,),
      in_specs=[
          pl.BlockSpec((NCORE, BN, HALF), lambda i: (0, i, 0)),
          pl.BlockSpec((BN, HID), lambda i: (i, 0)),
          pl.BlockSpec((BN, HALF), lambda i: (i, 0)),
          pl.BlockSpec((1, HID), lambda i: (0, 0)),
          pl.BlockSpec((HID, 5), lambda i: (0, 0)),
          pl.BlockSpec((1, 5), lambda i: (0, 0)),
      ],
      out_specs=pl.BlockSpec((1, 5), lambda i: (0, 0)),
      out_shape=jax.ShapeDtypeStruct((1, 5), jnp.float32),
      scratch_shapes=[pltpu.VMEM((8, 128), jnp.float32)],
  )(acc2, self2, dish, b2, wfc, bfc)


@jax.jit
def _run(x, edge_index, W1, b1, W2, b2, Wfc, bfc):
  src = edge_index[0].astype(jnp.int32)
  dst = edge_index[1].astype(jnp.int32)
  pad_src = (jnp.arange(PAD, dtype=jnp.int32) * 2621) % N
  pad_dst = N + (jnp.arange(PAD, dtype=jnp.int32) % 16)
  srcp = jnp.concatenate([src, pad_src])
  srcs2 = jnp.stack([srcp, srcp + N]).reshape(NCORE, ER, 128)
  dstp = jnp.concatenate([dst, pad_dst]).reshape(ER, 128)

  zeros = jnp.zeros((ZROWS, HALF), jnp.float32)
  ones = jnp.ones((128, HALF), jnp.float32)

  degh = _deg_call(dstp, zeros, ones)
  ycat, self1, dish = _tc_dense1(x, degh, W1)
  acc1 = _msg_call(ycat.reshape(NCORE * N, HALF), srcs2, dstp, zeros)
  ycat2, self2 = _tc_dense2(acc1, self1, dish, W2, b1.reshape(1, HID))
  acc2 = _msg_call(ycat2.reshape(NCORE * N, HALF), srcs2, dstp, zeros)
  return _tc_final(acc2, self2, dish, b2.reshape(1, HID), Wfc,
                   bfc.reshape(1, 5))


def kernel(x, edge_index, W1, b1, W2, b2, Wfc, bfc):
  return _run(x, edge_index, W1, b1, W2, b2, Wfc, bfc)


# speedup vs baseline: 31.5895x; 31.5895x over previous
import functools

import jax
import jax.numpy as jnp
from jax import lax
from jax.experimental import pallas as pl
from jax.experimental.pallas import tpu as pltpu
from jax.experimental.pallas import tpu_sc as plsc

N = 100000
E = 1600000
IN_F = 8
HID = 32
HALF = 16
NCORE = 2
NSUB = 16

NP = 100096
NV = NP
ZROWS = NV // NSUB
WROWS = NV // NSUB

EP = 1638400
PAD = EP - E
ER = EP // 128

MS_SLAB = 4
MS_RPS = ER // NSUB
MS_NSLAB = MS_RPS // MS_SLAB

DG_SLAB = 8
DG_RPS = ER // (NCORE * NSUB)
DG_NSLAB = DG_RPS // DG_SLAB

BN = 2000
GRID = N // BN

_mesh = plsc.VectorSubcoreMesh(core_axis_name="c", subcore_axis_name="s")


def _deg_body(dstp, zeros, ones, degh, didx, ones_v, degS, sem_i):
  c = lax.axis_index("c")
  s = lax.axis_index("s")
  pltpu.sync_copy(zeros, degS.at[pl.ds(s * ZROWS, ZROWS)])
  pltpu.sync_copy(ones, ones_v)
  row0 = (c * NSUB + s) * DG_RPS

  def idx_start(slab, b):
    pltpu.async_copy(
        dstp.at[pl.ds(row0 + slab * DG_SLAB, DG_SLAB)], didx.at[b],
        sem_i.at[b])

  def idx_wait(slab, b):
    pltpu.make_async_copy(
        dstp.at[pl.ds(row0 + slab * DG_SLAB, DG_SLAB)], didx.at[b],
        sem_i.at[b]).wait()

  idx_start(0, 0)
  plsc.subcore_barrier()

  def outer(g, carry):
    for b in (0, 1):
      i = g * 2 + b
      idx_wait(i, b)

      @pl.when(i + 1 < DG_NSLAB)
      def _():
        idx_start(i + 1, 1 - b)

      for j in range(DG_SLAB):
        pltpu.sync_copy(ones_v, degS.at[didx.at[b, j]], add=True)
    return carry

  lax.fori_loop(0, DG_NSLAB // 2, outer, 0)
  plsc.subcore_barrier()
  pltpu.sync_copy(degS.at[pl.ds(s * WROWS, WROWS)],
                  degh.at[c, pl.ds(s * WROWS, WROWS)])


_deg_call = pl.kernel(
    _deg_body,
    out_type=jax.ShapeDtypeStruct((NCORE, NP, HALF), jnp.float32),
    mesh=_mesh,
    compiler_params=pltpu.CompilerParams(use_tc_tiling_on_sc=False),
    scratch_types=[
        pltpu.VMEM((2, DG_SLAB, 128), jnp.int32),
        pltpu.VMEM((128, HALF), jnp.float32),
        pltpu.VMEM_SHARED((NV, HALF), jnp.float32),
        pltpu.SemaphoreType.DMA((2,)),
    ],
)


def _msg_body(ycat, srcs2, dstp, zeros, acc, sidx, didx, rows, accS,
              sem_g, sem_i):
  c = lax.axis_index("c")
  s = lax.axis_index("s")
  pltpu.sync_copy(zeros, accS.at[pl.ds(s * ZROWS, ZROWS)])
  row0 = s * MS_RPS

  def idx_start(slab, b):
    r0 = row0 + slab * MS_SLAB
    pltpu.async_copy(srcs2.at[c, pl.ds(r0, MS_SLAB)], sidx.at[b],
                     sem_i.at[b])
    pltpu.async_copy(dstp.at[pl.ds(r0, MS_SLAB)], didx.at[b], sem_i.at[b])

  def idx_wait(slab, b):
    r0 = row0 + slab * MS_SLAB
    pltpu.make_async_copy(srcs2.at[c, pl.ds(r0, MS_SLAB)], sidx.at[b],
                          sem_i.at[b]).wait()
    pltpu.make_async_copy(dstp.at[pl.ds(r0, MS_SLAB)], didx.at[b],
                          sem_i.at[b]).wait()

  def fire_gathers(b):
    for j in range(MS_SLAB):
      pltpu.async_copy(ycat.at[sidx.at[b, j]],
                       rows.at[b, pl.ds(j * 128, 128)], sem_g.at[b])

  def drain_gathers(b):
    for j in range(MS_SLAB):
      pltpu.make_async_copy(ycat.at[sidx.at[b, j]],
                            rows.at[b, pl.ds(j * 128, 128)],
                            sem_g.at[b]).wait()

  def scatters(b):
    for j in range(MS_SLAB):
      pltpu.sync_copy(rows.at[b, pl.ds(j * 128, 128)],
                      accS.at[didx.at[b, j]], add=True)

  idx_start(0, 0)
  idx_wait(0, 0)
  fire_gathers(0)
  idx_start(1, 1)
  plsc.subcore_barrier()

  def outer(g, carry):
    for b in (0, 1):
      i = g * 2 + b
      drain_gathers(b)

      @pl.when(i + 1 < MS_NSLAB)
      def _():
        idx_wait(i + 1, 1 - b)
        fire_gathers(1 - b)

      scatters(b)

      @pl.when(i + 2 < MS_NSLAB)
      def _():
        idx_start(i + 2, b)
    return carry

  lax.fori_loop(0, MS_NSLAB // 2, outer, 0)
  plsc.subcore_barrier()
  pltpu.sync_copy(accS.at[pl.ds(s * WROWS, WROWS)],
                  acc.at[c, pl.ds(s * WROWS, WROWS)])


_msg_call = pl.kernel(
    _msg_body,
    out_type=jax.ShapeDtypeStruct((NCORE, NP, HALF), jnp.float32),
    mesh=_mesh,
    compiler_params=pltpu.CompilerParams(use_tc_tiling_on_sc=False),
    scratch_types=[
        pltpu.VMEM((2, MS_SLAB, 128), jnp.int32),
        pltpu.VMEM((2, MS_SLAB, 128), jnp.int32),
        pltpu.VMEM((2, MS_SLAB * 128, HALF), jnp.float32),
        pltpu.VMEM_SHARED((NV, HALF), jnp.float32),
        pltpu.SemaphoreType.DMA((2,)),
        pltpu.SemaphoreType.DMA((2,)),
    ],
)


def _dense1_body(x, degh, w1, ycat, self1, dish):
  deg = degh[0][:, 0:1] + degh[1][:, 0:1] + 1.0
  dis = lax.rsqrt(deg)
  xw = jnp.dot(x[...], w1[...], preferred_element_type=jnp.float32)
  y = xw * dis
  ycat[...] = jnp.concatenate([y[:, :HALF][None], y[:, HALF:][None]], axis=0)
  self1[...] = y * dis
  dish[...] = jnp.broadcast_to(dis, (BN, HALF))


def _tc_dense1(x, degh, w1):
  return pl.pallas_call(
      _dense1_body,
      grid=(GRID,),
      in_specs=[
          pl.BlockSpec((BN, IN_F), lambda i: (i, 0)),
          pl.BlockSpec((NCORE, BN, HALF), lambda i: (0, i, 0)),
          pl.BlockSpec((IN_F, HID), lambda i: (0, 0)),
      ],
      out_specs=[
          pl.BlockSpec((NCORE, BN, HALF), lambda i: (0, i, 0)),
          pl.BlockSpec((BN, HID), lambda i: (i, 0)),
          pl.BlockSpec((BN, HALF), lambda i: (i, 0)),
      ],
      out_shape=[
          jax.ShapeDtypeStruct((NCORE, N, HALF), jnp.float32),
          jax.ShapeDtypeStruct((N, HID), jnp.float32),
          jax.ShapeDtypeStruct((N, HALF), jnp.float32),
      ],
  )(x, degh, w1)


def _dense2_body(acc1, self1, dish, w2, b1, ycat2, self2):
  dis = jnp.concatenate([dish[...], dish[...]], axis=1)
  accfull = jnp.concatenate([acc1[0], acc1[1]], axis=1)
  h1 = jnp.maximum(dis * accfull + self1[...] + b1[...], 0.0)
  xw2 = jnp.dot(h1, w2[...], preferred_element_type=jnp.float32)
  y2 = xw2 * dis
  ycat2[...] = jnp.concatenate([y2[:, :HALF][None], y2[:, HALF:][None]],
                               axis=0)
  self2[...] = y2 * dis


def _tc_dense2(acc1, self1, dish, w2, b1):
  return pl.pallas_call(
      _dense2_body,
      grid=(GRID,),
      in_specs=[
          pl.BlockSpec((NCORE, BN, HALF), lambda i: (0, i, 0)),
          pl.BlockSpec((BN, HID), lambda i: (i, 0)),
          pl.BlockSpec((BN, HALF), lambda i: (i, 0)),
          pl.BlockSpec((HID, HID), lambda i: (0, 0)),
          pl.BlockSpec((1, HID), lambda i: (0, 0)),
      ],
      out_specs=[
          pl.BlockSpec((NCORE, BN, HALF), lambda i: (0, i, 0)),
          pl.BlockSpec((BN, HID), lambda i: (i, 0)),
      ],
      out_shape=[
          jax.ShapeDtypeStruct((NCORE, N, HALF), jnp.float32),
          jax.ShapeDtypeStruct((N, HID), jnp.float32),
      ],
  )(acc1, self1, dish, w2, b1)


def _final_body(acc2, self2, dish, b2, wfc, bfc, out, scr):
  i = pl.program_id(0)
  dis = jnp.concatenate([dish[...], dish[...]], axis=1)
  accfull = jnp.concatenate([acc2[0], acc2[1]], axis=1)
  h2 = jnp.maximum(dis * accfull + self2[...] + b2[...], 0.0)
  bmax = jnp.max(h2, axis=0, keepdims=True)
  prev = jnp.where(i == 0, jnp.full((1, HID), -jnp.inf, jnp.float32),
                   scr[0:1, 0:HID])
  scr[0:1, 0:HID] = jnp.maximum(bmax, prev)

  @pl.when(i == GRID - 1)
  def _():
    pooled = scr[0:1, 0:HID]
    logits = jnp.sum(pooled.reshape(HID, 1) * wfc[...], axis=0,
                     keepdims=True) + bfc[...]
    m = jnp.max(logits, axis=1, keepdims=True)
    z = logits - m
    out[...] = z - jnp.log(jnp.sum(jnp.exp(z), axis=1, keepdims=True))


def _tc_final(acc2, self2, dish, b2, wfc, bfc):
  return pl.pallas_call(
      _final_body,
      grid=(GRID,),
      in_specs=[
          pl.BlockSpec((NCORE, BN, HALF), lambda i: (0, i, 0)),
          pl.BlockSpec((BN, HID), lambda i: (i, 0)),
          pl.BlockSpec((BN, HALF), lambda i: (i, 0)),
          pl.BlockSpec((1, HID), lambda i: (0, 0)),
          pl.BlockSpec((HID, 5), lambda i: (0, 0)),
          pl.BlockSpec((1, 5), lambda i: (0, 0)),
      ],
      out_specs=pl.BlockSpec((1, 5), lambda i: (0, 0)),
      out_shape=jax.ShapeDtypeStruct((1, 5), jnp.float32),
      scratch_shapes=[pltpu.VMEM((8, 128), jnp.float32)],
  )(acc2, self2, dish, b2, wfc, bfc)


@jax.jit
def _run(x, edge_index, W1, b1, W2, b2, Wfc, bfc):
  src = edge_index[0].astype(jnp.int32)
  dst = edge_index[1].astype(jnp.int32)
  pad_src = (jnp.arange(PAD, dtype=jnp.int32) * 2621) % N
  pad_dst = N + (jnp.arange(PAD, dtype=jnp.int32) % 16)
  srcp = jnp.concatenate([src, pad_src])
  srcs2 = jnp.stack([srcp, srcp + N]).reshape(NCORE, ER, 128)
  dstp = jnp.concatenate([dst, pad_dst]).reshape(ER, 128)

  zeros = jnp.zeros((ZROWS, HALF), jnp.float32)
  ones = jnp.ones((128, HALF), jnp.float32)

  degh = _deg_call(dstp, zeros, ones)
  ycat, self1, dish = _tc_dense1(x, degh, W1)
  acc1 = _msg_call(ycat.reshape(NCORE * N, HALF), srcs2, dstp, zeros)
  ycat2, self2 = _tc_dense2(acc1, self1, dish, W2, b1.reshape(1, HID))
  acc2 = _msg_call(ycat2.reshape(NCORE * N, HALF), srcs2, dstp, zeros)
  return _tc_final(acc2, self2, dish, b2.reshape(1, HID), Wfc,
                   bfc.reshape(1, 5))


def kernel(x, edge_index, W1, b1, W2, b2, Wfc, bfc):
  return _run(x, edge_index, W1, b1, W2, b2, Wfc, bfc)


# speedup vs baseline: 31.9987x; 1.0130x over previous
import functools

import jax
import jax.numpy as jnp
from jax import lax
from jax.experimental import pallas as pl
from jax.experimental.pallas import tpu as pltpu
from jax.experimental.pallas import tpu_sc as plsc

N = 100000
E = 1600000
IN_F = 8
HID = 32
HALF = 16
NCORE = 2
NSUB = 16

NP = 100096
NV = NP
ZROWS = NV // NSUB
WROWS = NV // NSUB

EP = 1638400
PAD = EP - E
ER = EP // 128

MS_SLAB = 4
MS_RPS = ER // NSUB
MS_NSLAB = MS_RPS // MS_SLAB

DG_SLAB = 4
DG_RPS = ER // (NCORE * NSUB)
DG_NSLAB = DG_RPS // DG_SLAB

BN = 2000
GRID = N // BN

_mesh = plsc.VectorSubcoreMesh(core_axis_name="c", subcore_axis_name="s")


def _deg_body(dstp, zeros, ones, degh, didx, ones_v, degS, sem_i, sem_s):
  c = lax.axis_index("c")
  s = lax.axis_index("s")
  pltpu.sync_copy(zeros, degS.at[pl.ds(s * ZROWS, ZROWS)])
  pltpu.sync_copy(ones, ones_v)
  row0 = (c * NSUB + s) * DG_RPS

  def idx_start(slab, q):
    pltpu.async_copy(
        dstp.at[pl.ds(row0 + slab * DG_SLAB, DG_SLAB)], didx.at[q],
        sem_i.at[q])

  def idx_wait(slab, q):
    pltpu.make_async_copy(
        dstp.at[pl.ds(row0 + slab * DG_SLAB, DG_SLAB)], didx.at[q],
        sem_i.at[q]).wait()

  def fire_scatters(b, q):
    for j in range(DG_SLAB):
      pltpu.async_copy(ones_v, degS.at[didx.at[q, j]], sem_s.at[b],
                       add=True)

  def drain_scatters(b, q):
    for j in range(DG_SLAB):
      pltpu.make_async_copy(ones_v, degS.at[didx.at[q, j]],
                            sem_s.at[b]).wait()

  idx_start(0, 0)
  idx_start(1, 1)
  plsc.subcore_barrier()

  def outer(g, carry):
    for k in range(4):
      i = g * 4 + k
      b = k % 2
      q = k % 4
      idx_wait(i, q)
      if k < 2:
        @pl.when(g > 0)
        def _():
          drain_scatters(b, (k - 2) % 4)
      else:
        drain_scatters(b, (k - 2) % 4)
      fire_scatters(b, q)

      @pl.when(i + 2 < DG_NSLAB)
      def _():
        idx_start(i + 2, (k + 2) % 4)
    return carry

  lax.fori_loop(0, DG_NSLAB // 4, outer, 0)
  drain_scatters(0, 2)
  drain_scatters(1, 3)
  plsc.subcore_barrier()
  pltpu.sync_copy(degS.at[pl.ds(s * WROWS, WROWS)],
                  degh.at[c, pl.ds(s * WROWS, WROWS)])


_deg_call = pl.kernel(
    _deg_body,
    out_type=jax.ShapeDtypeStruct((NCORE, NP, HALF), jnp.float32),
    mesh=_mesh,
    compiler_params=pltpu.CompilerParams(use_tc_tiling_on_sc=False),
    scratch_types=[
        pltpu.VMEM((4, DG_SLAB, 128), jnp.int32),
        pltpu.VMEM((128, HALF), jnp.float32),
        pltpu.VMEM_SHARED((NV, HALF), jnp.float32),
        pltpu.SemaphoreType.DMA((4,)),
        pltpu.SemaphoreType.DMA((2,)),
    ],
)


def _msg_body(ycat, srcs2, dstp, zeros, acc, sidx, didx, rows, accS,
              sem_g, sem_i, sem_s):
  c = lax.axis_index("c")
  s = lax.axis_index("s")
  pltpu.sync_copy(zeros, accS.at[pl.ds(s * ZROWS, ZROWS)])
  row0 = s * MS_RPS

  def idx_start(slab, q):
    r0 = row0 + slab * MS_SLAB
    pltpu.async_copy(srcs2.at[c, pl.ds(r0, MS_SLAB)], sidx.at[q],
                     sem_i.at[q])
    pltpu.async_copy(dstp.at[pl.ds(r0, MS_SLAB)], didx.at[q], sem_i.at[q])

  def idx_wait(slab, q):
    r0 = row0 + slab * MS_SLAB
    pltpu.make_async_copy(srcs2.at[c, pl.ds(r0, MS_SLAB)], sidx.at[q],
                          sem_i.at[q]).wait()
    pltpu.make_async_copy(dstp.at[pl.ds(r0, MS_SLAB)], didx.at[q],
                          sem_i.at[q]).wait()

  def fire_gathers(b, q):
    for j in range(MS_SLAB):
      pltpu.async_copy(ycat.at[sidx.at[q, j]],
                       rows.at[b, pl.ds(j * 128, 128)], sem_g.at[b])

  def drain_gathers(b, q):
    for j in range(MS_SLAB):
      pltpu.make_async_copy(ycat.at[sidx.at[q, j]],
                            rows.at[b, pl.ds(j * 128, 128)],
                            sem_g.at[b]).wait()

  def fire_scatters(b, q):
    for j in range(MS_SLAB):
      pltpu.async_copy(rows.at[b, pl.ds(j * 128, 128)],
                       accS.at[didx.at[q, j]], sem_s.at[b], add=True)

  def drain_scatters(b, q):
    for j in range(MS_SLAB):
      pltpu.make_async_copy(rows.at[b, pl.ds(j * 128, 128)],
                            accS.at[didx.at[q, j]], sem_s.at[b]).wait()

  idx_start(0, 0)
  idx_wait(0, 0)
  fire_gathers(0, 0)
  idx_start(1, 1)
  plsc.subcore_barrier()

  def outer(g, carry):
    for k in range(4):
      i = g * 4 + k
      b = k % 2
      q = k % 4
      drain_gathers(b, q)
      if k == 0:
        @pl.when(g > 0)
        def _():
          drain_scatters(1 - b, (k - 1) % 4)
      else:
        drain_scatters(1 - b, (k - 1) % 4)
      if k == 3:
        @pl.when(g < MS_NSLAB // 4 - 1)
        def _():
          idx_wait(i + 1, (k + 1) % 4)
          fire_gathers(1 - b, (k + 1) % 4)
      else:
        idx_wait(i + 1, (k + 1) % 4)
        fire_gathers(1 - b, (k + 1) % 4)
      fire_scatters(b, q)

      @pl.when(i + 2 < MS_NSLAB)
      def _():
        idx_start(i + 2, (k + 2) % 4)
    return carry

  lax.fori_loop(0, MS_NSLAB // 4, outer, 0)
  drain_scatters(1, 3)
  plsc.subcore_barrier()
  pltpu.sync_copy(accS.at[pl.ds(s * WROWS, WROWS)],
                  acc.at[c, pl.ds(s * WROWS, WROWS)])


_msg_call = pl.kernel(
    _msg_body,
    out_type=jax.ShapeDtypeStruct((NCORE, NP, HALF), jnp.float32),
    mesh=_mesh,
    compiler_params=pltpu.CompilerParams(use_tc_tiling_on_sc=False),
    scratch_types=[
        pltpu.VMEM((4, MS_SLAB, 128), jnp.int32),
        pltpu.VMEM((4, MS_SLAB, 128), jnp.int32),
        pltpu.VMEM((2, MS_SLAB * 128, HALF), jnp.float32),
        pltpu.VMEM_SHARED((NV, HALF), jnp.float32),
        pltpu.SemaphoreType.DMA((2,)),
        pltpu.SemaphoreType.DMA((4,)),
        pltpu.SemaphoreType.DMA((2,)),
    ],
)


def _dense1_body(x, degh, w1, ycat, self1, dish):
  deg = degh[0][:, 0:1] + degh[1][:, 0:1] + 1.0
  dis = lax.rsqrt(deg)
  xw = jnp.dot(x[...], w1[...], preferred_element_type=jnp.float32)
  y = xw * dis
  ycat[...] = jnp.concatenate([y[:, :HALF][None], y[:, HALF:][None]], axis=0)
  self1[...] = y * dis
  dish[...] = jnp.broadcast_to(dis, (BN, HALF))


def _tc_dense1(x, degh, w1):
  return pl.pallas_call(
      _dense1_body,
      grid=(GRID,),
      in_specs=[
          pl.BlockSpec((BN, IN_F), lambda i: (i, 0)),
          pl.BlockSpec((NCORE, BN, HALF), lambda i: (0, i, 0)),
          pl.BlockSpec((IN_F, HID), lambda i: (0, 0)),
      ],
      out_specs=[
          pl.BlockSpec((NCORE, BN, HALF), lambda i: (0, i, 0)),
          pl.BlockSpec((BN, HID), lambda i: (i, 0)),
          pl.BlockSpec((BN, HALF), lambda i: (i, 0)),
      ],
      out_shape=[
          jax.ShapeDtypeStruct((NCORE, N, HALF), jnp.float32),
          jax.ShapeDtypeStruct((N, HID), jnp.float32),
          jax.ShapeDtypeStruct((N, HALF), jnp.float32),
      ],
  )(x, degh, w1)


def _dense2_body(acc1, self1, dish, w2, b1, ycat2, self2):
  dis = jnp.concatenate([dish[...], dish[...]], axis=1)
  accfull = jnp.concatenate([acc1[0], acc1[1]], axis=1)
  h1 = jnp.maximum(dis * accfull + self1[...] + b1[...], 0.0)
  xw2 = jnp.dot(h1, w2[...], preferred_element_type=jnp.float32)
  y2 = xw2 * dis
  ycat2[...] = jnp.concatenate([y2[:, :HALF][None], y2[:, HALF:][None]],
                               axis=0)
  self2[...] = y2 * dis


def _tc_dense2(acc1, self1, dish, w2, b1):
  return pl.pallas_call(
      _dense2_body,
      grid=(GRID,),
      in_specs=[
          pl.BlockSpec((NCORE, BN, HALF), lambda i: (0, i, 0)),
          pl.BlockSpec((BN, HID), lambda i: (i, 0)),
          pl.BlockSpec((BN, HALF), lambda i: (i, 0)),
          pl.BlockSpec((HID, HID), lambda i: (0, 0)),
          pl.BlockSpec((1, HID), lambda i: (0, 0)),
      ],
      out_specs=[
          pl.BlockSpec((NCORE, BN, HALF), lambda i: (0, i, 0)),
          pl.BlockSpec((BN, HID), lambda i: (i, 0)),
      ],
      out_shape=[
          jax.ShapeDtypeStruct((NCORE, N, HALF), jnp.float32),
          jax.ShapeDtypeStruct((N, HID), jnp.float32),
      ],
  )(acc1, self1, dish, w2, b1)


def _final_body(acc2, self2, dish, b2, wfc, bfc, out, scr):
  i = pl.program_id(0)
  dis = jnp.concatenate([dish[...], dish[...]], axis=1)
  accfull = jnp.concatenate([acc2[0], acc2[1]], axis=1)
  h2 = jnp.maximum(dis * accfull + self2[...] + b2[...], 0.0)
  bmax = jnp.max(h2, axis=0, keepdims=True)
  prev = jnp.where(i == 0, jnp.full((1, HID), -jnp.inf, jnp.float32),
                   scr[0:1, 0:HID])
  scr[0:1, 0:HID] = jnp.maximum(bmax, prev)

  @pl.when(i == GRID - 1)
  def _():
    pooled = scr[0:1, 0:HID]
    logits = jnp.sum(pooled.reshape(HID, 1) * wfc[...], axis=0,
                     keepdims=True) + bfc[...]
    m = jnp.max(logits, axis=1, keepdims=True)
    z = logits - m
    out[...] = z - jnp.log(jnp.sum(jnp.exp(z), axis=1, keepdims=True))


def _tc_final(acc2, self2, dish, b2, wfc, bfc):
  return pl.pallas_call(
      _final_body,
      grid=(GRID,),
      in_specs=[
          pl.BlockSpec((NCORE, BN, HALF), lambda i: (0, i, 0)),
          pl.BlockSpec((BN, HID), lambda i: (i, 0)),
          pl.BlockSpec((BN, HALF), lambda i: (i, 0)),
          pl.BlockSpec((1, HID), lambda i: (0, 0)),
          pl.BlockSpec((HID, 5), lambda i: (0, 0)),
          pl.BlockSpec((1, 5), lambda i: (0, 0)),
      ],
      out_specs=pl.BlockSpec((1, 5), lambda i: (0, 0)),
      out_shape=jax.ShapeDtypeStruct((1, 5), jnp.float32),
      scratch_shapes=[pltpu.VMEM((8, 128), jnp.float32)],
  )(acc2, self2, dish, b2, wfc, bfc)


@jax.jit
def _run(x, edge_index, W1, b1, W2, b2, Wfc, bfc):
  src = edge_index[0].astype(jnp.int32)
  dst = edge_index[1].astype(jnp.int32)
  pad_src = (jnp.arange(PAD, dtype=jnp.int32) * 2621) % N
  pad_dst = N + (jnp.arange(PAD, dtype=jnp.int32) % 16)
  srcp = jnp.concatenate([src, pad_src])
  srcs2 = jnp.stack([srcp, srcp + N]).reshape(NCORE, ER, 128)
  dstp = jnp.concatenate([dst, pad_dst]).reshape(ER, 128)

  zeros = jnp.zeros((ZROWS, HALF), jnp.float32)
  ones = jnp.ones((128, HALF), jnp.float32)

  degh = _deg_call(dstp, zeros, ones)
  ycat, self1, dish = _tc_dense1(x, degh, W1)
  acc1 = _msg_call(ycat.reshape(NCORE * N, HALF), srcs2, dstp, zeros)
  ycat2, self2 = _tc_dense2(acc1, self1, dish, W2, b1.reshape(1, HID))
  acc2 = _msg_call(ycat2.reshape(NCORE * N, HALF), srcs2, dstp, zeros)
  return _tc_final(acc2, self2, dish, b2.reshape(1, HID), Wfc,
                   bfc.reshape(1, 5))


def kernel(x, edge_index, W1, b1, W2, b2, Wfc, bfc):
  return _run(x, edge_index, W1, b1, W2, b2, Wfc, bfc)
